# baseline (speedup 1.0000x reference)
import jax
import jax.numpy as jnp
from jax.experimental import pallas as pl
from jax.experimental.pallas import tpu as pltpu

_ROWS = 256


def _mix_body(tok_ref, pos_ref, m_ref, me_ref, tw_ref, pw_ref, mw_ref, out_ref):
    tw = tw_ref[0, 0]
    pw = pw_ref[0, 0]
    mw = mw_ref[0, 0]
    masked_row = mw * me_ref[0, :]
    m = m_ref[0, 0, :][:, None]
    out_ref[...] = tw * tok_ref[...] + pw * pos_ref[...] + m * masked_row[None, :]


def kernel(token_embeds, mask_embeds, position_embeds, mask_inds,
           token_weight, mask_weight, position_weight):
    B, S, D = token_embeds.shape
    N = B * S
    R = _ROWS
    nblk = N // R

    tok2 = token_embeds.reshape(N, D)
    pos2 = position_embeds.reshape(N, D)
    maskf = mask_inds.reshape(nblk, 1, R).astype(jnp.float32)
    me2 = mask_embeds.reshape(1, D)
    tw2 = token_weight.reshape(1, 1)
    pw2 = position_weight.reshape(1, 1)
    mw2 = mask_weight.reshape(1, 1)

    out = pl.pallas_call(
        _mix_body,
        grid=(nblk,),
        in_specs=[
            pl.BlockSpec((R, D), lambda i: (i, 0)),
            pl.BlockSpec((R, D), lambda i: (i, 0)),
            pl.BlockSpec((1, 1, R), lambda i: (i, 0, 0)),
            pl.BlockSpec((1, D), lambda i: (0, 0)),
            pl.BlockSpec((1, 1), lambda i: (0, 0)),
            pl.BlockSpec((1, 1), lambda i: (0, 0)),
            pl.BlockSpec((1, 1), lambda i: (0, 0)),
        ],
        out_specs=pl.BlockSpec((R, D), lambda i: (i, 0)),
        out_shape=jax.ShapeDtypeStruct((N, D), jnp.float32),
        compiler_params=pltpu.CompilerParams(
            dimension_semantics=("arbitrary",),
        ),
    )(tok2, pos2, maskf, me2, tw2, pw2, mw2)
    return out.reshape(B, S, D)


# speedup vs baseline: 1.0366x; 1.0366x over previous
import jax
import jax.numpy as jnp
from jax.experimental import pallas as pl
from jax.experimental.pallas import tpu as pltpu

_ROWS = 512


def _mix_body(tok_ref, pos_ref, m_ref, me_ref, tw_ref, pw_ref, mw_ref, out_ref):
    tw = tw_ref[0, 0]
    pw = pw_ref[0, 0]
    mw = mw_ref[0, 0]
    masked_row = mw * me_ref[0, :]
    m = m_ref[0, 0, :][:, None]
    out_ref[...] = tw * tok_ref[...] + pw * pos_ref[...] + m * masked_row[None, :]


def kernel(token_embeds, mask_embeds, position_embeds, mask_inds,
           token_weight, mask_weight, position_weight):
    B, S, D = token_embeds.shape
    N = B * S
    R = _ROWS
    nblk = N // R

    tok2 = token_embeds.reshape(N, D)
    pos2 = position_embeds.reshape(N, D)
    maskf = mask_inds.reshape(nblk, 1, R).astype(jnp.float32)
    me2 = mask_embeds.reshape(1, D)
    tw2 = token_weight.reshape(1, 1)
    pw2 = position_weight.reshape(1, 1)
    mw2 = mask_weight.reshape(1, 1)

    out = pl.pallas_call(
        _mix_body,
        grid=(nblk,),
        in_specs=[
            pl.BlockSpec((R, D), lambda i: (i, 0)),
            pl.BlockSpec((R, D), lambda i: (i, 0)),
            pl.BlockSpec((1, 1, R), lambda i: (i, 0, 0)),
            pl.BlockSpec((1, D), lambda i: (0, 0)),
            pl.BlockSpec((1, 1), lambda i: (0, 0)),
            pl.BlockSpec((1, 1), lambda i: (0, 0)),
            pl.BlockSpec((1, 1), lambda i: (0, 0)),
        ],
        out_specs=pl.BlockSpec((R, D), lambda i: (i, 0)),
        out_shape=jax.ShapeDtypeStruct((N, D), jnp.float32),
        compiler_params=pltpu.CompilerParams(
            dimension_semantics=("arbitrary",),
        ),
    )(tok2, pos2, maskf, me2, tw2, pw2, mw2)
    return out.reshape(B, S, D)
